# Initial kernel scaffold; baseline (speedup 1.0000x reference)
import jax
import jax.numpy as jnp
from jax.experimental import pallas as pl


def kernel(input_x, table):
    raise NotImplementedError("write your pallas kernel here")


# speedup vs baseline: 1.6031x; 1.6031x over previous
import functools

import jax
import jax.numpy as jnp
from jax import lax
from jax.experimental import pallas as pl
from jax.experimental.pallas import tpu as pltpu
from jax.experimental.pallas import tpu_sc as plsc

D = 32
NC = 2
NS = 16
NW = NC * NS
CHUNK = 1024


def _gather_body(idx_hbm, table_hbm, out_hbm, idx_v, rows_v, sem):
    wid = lax.axis_index("s") * NC + lax.axis_index("c")
    n_per_w = idx_hbm.shape[0] // NW
    base = wid * n_per_w
    nchunks = n_per_w // CHUNK

    def step(c, carry):
        off = base + c * CHUNK
        pltpu.sync_copy(idx_hbm.at[pl.ds(off, CHUNK)], idx_v)
        pltpu.async_copy(table_hbm.at[idx_v], rows_v, sem).wait()
        pltpu.sync_copy(rows_v, out_hbm.at[pl.ds(off, CHUNK)])
        return carry

    lax.fori_loop(0, nchunks, step, 0)


def kernel(input_x, table):
    B, L = input_x.shape
    N = B * L
    idx = input_x.reshape(N).astype(jnp.int32)
    table = table.astype(jnp.float32)

    mesh = plsc.VectorSubcoreMesh(core_axis_name="c", subcore_axis_name="s")
    call = functools.partial(
        pl.kernel,
        mesh=mesh,
        out_type=jax.ShapeDtypeStruct((N, D), jnp.float32),
        scratch_types=[
            pltpu.VMEM((CHUNK,), jnp.int32),
            pltpu.VMEM((CHUNK, D), jnp.float32),
            pltpu.SemaphoreType.DMA,
        ],
        compiler_params=pltpu.CompilerParams(use_tc_tiling_on_sc=False),
    )(_gather_body)
    out = call(idx, table)
    return out.reshape(B, L, D)


# speedup vs baseline: 1.6769x; 1.0460x over previous
import functools

import jax
import jax.numpy as jnp
from jax import lax
from jax.experimental import pallas as pl
from jax.experimental.pallas import tpu as pltpu
from jax.experimental.pallas import tpu_sc as plsc

D = 32
NC = 2
NS = 16
NW = NC * NS
CHUNK = 1024
NBUF = 2


def _gather_body(idx_hbm, table_hbm, out_hbm,
                 idx0, idx1, rows0, rows1, gs0, gs1, ws0, ws1):
    wid = lax.axis_index("s") * NC + lax.axis_index("c")
    n_per_w = idx_hbm.shape[0] // NW
    base = wid * n_per_w
    nchunks = n_per_w // CHUNK
    nouter = nchunks // NBUF

    idxv = (idx0, idx1)
    rowsv = (rows0, rows1)
    gs = (gs0, gs1)
    ws = (ws0, ws1)

    def start_gather(c, b):
        off = base + c * CHUNK
        pltpu.sync_copy(idx_hbm.at[pl.ds(off, CHUNK)], idxv[b])
        pltpu.async_copy(table_hbm.at[idxv[b]], rowsv[b], gs[b])

    def wait_gather(b):
        pltpu.make_async_copy(table_hbm.at[idxv[b]], rowsv[b], gs[b]).wait()

    def start_write(c, b):
        off = base + c * CHUNK
        pltpu.async_copy(rowsv[b], out_hbm.at[pl.ds(off, CHUNK)], ws[b])

    def wait_write(c, b):
        off = base + c * CHUNK
        pltpu.make_async_copy(rowsv[b], out_hbm.at[pl.ds(off, CHUNK)], ws[b]).wait()

    for b in range(NBUF):
        start_gather(b, b)

    def iteration(i, issue_next):
        for b in range(NBUF):
            c = i * NBUF + b
            wait_gather(b)
            start_write(c, b)
        if issue_next:
            for b in range(NBUF):
                c = i * NBUF + b
                wait_write(c, b)
                start_gather(c + NBUF, b)

    def loop_body(i, carry):
        iteration(i, issue_next=True)
        return carry

    lax.fori_loop(0, nouter - 1, loop_body, 0)
    iteration(nouter - 1, issue_next=False)
    for b in range(NBUF):
        c = (nouter - 1) * NBUF + b
        wait_write(c, b)


def kernel(input_x, table):
    B, L = input_x.shape
    N = B * L
    idx = input_x.reshape(N).astype(jnp.int32)
    table = table.astype(jnp.float32)

    mesh = plsc.VectorSubcoreMesh(core_axis_name="c", subcore_axis_name="s")
    call = functools.partial(
        pl.kernel,
        mesh=mesh,
        out_type=jax.ShapeDtypeStruct((N, D), jnp.float32),
        scratch_types=[
            pltpu.VMEM((CHUNK,), jnp.int32),
            pltpu.VMEM((CHUNK,), jnp.int32),
            pltpu.VMEM((CHUNK, D), jnp.float32),
            pltpu.VMEM((CHUNK, D), jnp.float32),
            pltpu.SemaphoreType.DMA,
            pltpu.SemaphoreType.DMA,
            pltpu.SemaphoreType.DMA,
            pltpu.SemaphoreType.DMA,
        ],
        compiler_params=pltpu.CompilerParams(use_tc_tiling_on_sc=False),
    )(_gather_body)
    out = call(idx, table)
    return out.reshape(B, L, D)


# speedup vs baseline: 1.8063x; 1.0772x over previous
import functools

import jax
import jax.numpy as jnp
from jax import lax
from jax.experimental import pallas as pl
from jax.experimental.pallas import tpu as pltpu
from jax.experimental.pallas import tpu_sc as plsc

D = 32
NC = 2
NS = 16
NW = NC * NS
V = 1_000_000
RT = 7813
RT_FULL = 7812
ABLK = 244
B_BATCH = 4096
L_SEQ = 512


def _maybe_when(cond, fn):
    if isinstance(cond, bool):
        if cond:
            fn()
    else:
        pl.when(cond)(fn)


def _format_body(tableT, idx_nat, aux, idxlin,
                 tiles, blk, ibufa, ilin, ts0, ts1, ws0, ws1, qs):
    wid = lax.axis_index("s") * NC + lax.axis_index("c")
    tsem = (ts0, ts1)
    wsem = (ws0, ws1)

    cc0 = jnp.arange(16, dtype=jnp.int32)
    ivec = (cc0 // 8, (cc0 + 16) // 8)
    svec = (cc0 % 8, (cc0 + 16) % 8)

    my_start = wid * ABLK

    def stage(jj, b):
        for i in range(4):
            pltpu.async_copy(
                tableT.at[pl.ds(i * 8, 8), pl.ds(jj * 128, 128)],
                tiles.at[b, i, :, pl.ds(0, 128)], tsem[b])

    def wait_stage(b):
        for i in range(4):
            pltpu.make_async_copy(
                tableT.at[pl.ds(0, 8), pl.ds(0, 128)],
                tiles.at[b, i, :, pl.ds(0, 128)], tsem[b]).wait()

    def transpose_block(b, nrows):
        def row4(r4, carry):
            vals = []
            for dr in range(4):
                rsp = lax.broadcast(r4 * 4 + dr, (16,))
                for h in range(2):
                    vals.append(
                        plsc.load_gather(tiles.at[b],
                                         [ivec[h], svec[h], rsp]))
            for dr in range(4):
                rr4 = r4 * 4 + dr
                for h in range(2):
                    blk[b, pl.ds(rr4 * D + h * 16, 16)] = vals[dr * 2 + h]
            return carry
        lax.fori_loop(0, nrows // 4, row4, 0)

    def start_write(jj, b):
        pltpu.async_copy(blk.at[b], aux.at[pl.ds(jj * 4096, 4096)], wsem[b])

    def wait_write(b):
        pltpu.make_async_copy(blk.at[b], aux.at[pl.ds(0, 4096)], wsem[b]).wait()

    stage(my_start, 0)
    stage(my_start + 1, 1)

    def pair(p, first):
        for b in range(2):
            k = p * 2 + b
            jj = my_start + k
            wait_stage(b)
            if not first:
                wait_write(b)
            transpose_block(b, 128)
            _maybe_when(k + 2 < ABLK, lambda b=b, jj=jj: stage(jj + 2, b))
            start_write(jj, b)

    pair(0, True)

    def loop_body(p, carry):
        pair(p, False)
        return carry

    lax.fori_loop(1, ABLK // 2, loop_body, 0)
    wait_write(0)
    wait_write(1)

    @pl.when(wid < 4)
    def _extra():
        jj = RT_FULL - 4 + wid
        stage(jj, 0)
        wait_stage(0)
        transpose_block(0, 128)
        start_write(jj, 0)
        wait_write(0)

    @pl.when(wid == NW - 1)
    def _tail():
        stage(wid - (NW - 1) + RT_FULL, 0)
        wait_stage(0)
        transpose_block(0, 64)
        pltpu.async_copy(blk.at[0, pl.ds(0, 2048)],
                         aux.at[pl.ds(RT_FULL * 4096, 2048)], ws0)
        pltpu.make_async_copy(blk.at[0, pl.ds(0, 2048)],
                              aux.at[pl.ds(RT_FULL * 4096, 2048)], ws0).wait()

    def group(g, carry):
        q = wid * 16 + g
        for j in range(4):
            pltpu.async_copy(
                idx_nat.at[pl.ds(q * 8, 8), pl.ds(j * 128, 128)],
                ibufa.at[j], qs)
        for j in range(4):
            pltpu.make_async_copy(
                idx_nat.at[pl.ds(0, 8), pl.ds(0, 128)],
                ibufa.at[j], qs).wait()
        for ap in range(8):
            for j in range(4):
                for seg in range(8):
                    ilin[pl.ds(ap * 512 + j * 128 + seg * 16, 16)] = (
                        ibufa[j, ap, pl.ds(seg * 16, 16)])
        pltpu.async_copy(ilin, idxlin.at[pl.ds(q * 4096, 4096)], qs)
        pltpu.make_async_copy(ilin, idxlin.at[pl.ds(q * 4096, 4096)], qs).wait()
        return carry

    lax.fori_loop(0, 16, group, 0)


def _gather_body(aux2d, idxlin, out, ibuf, g_buf, t_buf,
                 is0, is1, gs0, gs1, ws0, ws1):
    wid = lax.axis_index("s") * NC + lax.axis_index("c")
    isem = (is0, is1)
    gsem = (gs0, gs1)
    wsem = (ws0, ws1)

    base = wid * 128
    cc0 = jnp.arange(16, dtype=jnp.int32)

    def stage_idx(k, b):
        pltpu.async_copy(idxlin.at[pl.ds((base + k) * L_SEQ, L_SEQ)],
                         ibuf.at[b], isem[b])

    def wait_idx(b):
        pltpu.make_async_copy(idxlin.at[pl.ds(0, L_SEQ)],
                              ibuf.at[b], isem[b]).wait()

    def start_gather(b):
        pltpu.async_copy(aux2d.at[ibuf.at[b]], g_buf.at[b], gsem[b])

    def wait_gather(b):
        pltpu.make_async_copy(aux2d.at[ibuf.at[b]], g_buf.at[b],
                              gsem[b]).wait()

    def transpose_slab(b):
        def unit(v, carry):
            j = v // 8
            u0 = v % 8
            lv = cc0 + lax.broadcast(j * 128 + u0 * 16, (16,))
            for h in range(2):
                vals = [
                    plsc.load_gather(g_buf.at[b],
                                     [lv, lax.broadcast(c, (16,))])
                    for c in range(h * 16, h * 16 + 16)
                ]
                for i, c in enumerate(range(h * 16, h * 16 + 16)):
                    t_buf[b, (c // 8) * 4 + j, c % 8,
                          pl.ds(u0 * 16, 16)] = vals[i]
            return carry
        lax.fori_loop(0, 32, unit, 0)

    def start_write(k, b):
        pltpu.async_copy(t_buf.at[b], out.at[base + k], wsem[b])

    def wait_write(b):
        pltpu.make_async_copy(t_buf.at[b], out.at[base], wsem[b]).wait()

    stage_idx(0, 0)
    stage_idx(1, 1)
    wait_idx(0)
    start_gather(0)

    def pair(p, first):
        for b2 in range(2):
            k = p * 2 + b2
            wait_gather(b2)
            _maybe_when(k + 2 < 128, lambda b2=b2, k=k: stage_idx(k + 2, b2))

            def _next(b2=b2):
                wait_idx(1 - b2)
                start_gather(1 - b2)
            _maybe_when(k + 1 < 128, _next)
            if not first:
                wait_write(b2)
            transpose_slab(b2)
            start_write(k, b2)

    pair(0, True)

    def loop_body(p, carry):
        pair(p, False)
        return carry

    lax.fori_loop(1, 64, loop_body, 0)
    wait_write(0)
    wait_write(1)


def kernel(input_x, table):
    idx_nat = input_x.astype(jnp.int32)
    tableT = table.astype(jnp.float32).T

    mesh = plsc.VectorSubcoreMesh(core_axis_name="c", subcore_axis_name="s")

    fmt = functools.partial(
        pl.kernel,
        mesh=mesh,
        out_type=(
            jax.ShapeDtypeStruct((V * D,), jnp.float32),
            jax.ShapeDtypeStruct((B_BATCH * L_SEQ,), jnp.int32),
        ),
        scratch_types=[
            pltpu.VMEM((2, 4, 8, 131), jnp.float32),
            pltpu.VMEM((2, 4096), jnp.float32),
            pltpu.VMEM((4, 8, 128), jnp.int32),
            pltpu.VMEM((4096,), jnp.int32),
            pltpu.SemaphoreType.DMA,
            pltpu.SemaphoreType.DMA,
            pltpu.SemaphoreType.DMA,
            pltpu.SemaphoreType.DMA,
            pltpu.SemaphoreType.DMA,
        ],
        compiler_params=pltpu.CompilerParams(use_tc_tiling_on_sc=True, needs_layout_passes=False),
    )(_format_body)
    aux, idxlin = fmt(tableT, idx_nat)

    gat = functools.partial(
        pl.kernel,
        mesh=mesh,
        out_type=jax.ShapeDtypeStruct((B_BATCH, 16, 8, 128), jnp.float32),
        scratch_types=[
            pltpu.VMEM((2, L_SEQ), jnp.int32),
            pltpu.VMEM((2, L_SEQ, D), jnp.float32),
            pltpu.VMEM((2, 16, 8, 128), jnp.float32),
            pltpu.SemaphoreType.DMA,
            pltpu.SemaphoreType.DMA,
            pltpu.SemaphoreType.DMA,
            pltpu.SemaphoreType.DMA,
            pltpu.SemaphoreType.DMA,
            pltpu.SemaphoreType.DMA,
        ],
        compiler_params=pltpu.CompilerParams(use_tc_tiling_on_sc=False, needs_layout_passes=False),
    )(_gather_body)
    out_tiled = gat(aux.reshape(V, D), idxlin)

    out5 = out_tiled.reshape(B_BATCH, 4, 4, 8, 128)
    return out5.transpose(0, 2, 4, 1, 3).reshape(B_BATCH, L_SEQ, D)


# speedup vs baseline: 1.8084x; 1.0011x over previous
import functools

import jax
import jax.numpy as jnp
from jax import lax
from jax.experimental import pallas as pl
from jax.experimental.pallas import tpu as pltpu
from jax.experimental.pallas import tpu_sc as plsc

D = 32
NC = 2
NS = 16
NW = NC * NS
V = 1_000_000
RT = 7813
RT_FULL = 7812
ABLK = 244
B_BATCH = 4096
L_SEQ = 512


def _maybe_when(cond, fn):
    if isinstance(cond, bool):
        if cond:
            fn()
    else:
        pl.when(cond)(fn)


def _format_body(tableT, idx_nat, aux, idxlin,
                 tiles, blk, ibufa, ilin, ts0, ts1, ws0, ws1, qs):
    wid = lax.axis_index("s") * NC + lax.axis_index("c")
    tsem = (ts0, ts1)
    wsem = (ws0, ws1)

    cc0 = jnp.arange(16, dtype=jnp.int32)
    ivec = (cc0 // 8, (cc0 + 16) // 8)
    svec = (cc0 % 8, (cc0 + 16) % 8)

    my_start = wid * ABLK

    def stage(jj, b):
        for i in range(4):
            pltpu.async_copy(
                tableT.at[pl.ds(i * 8, 8), pl.ds(jj * 128, 128)],
                tiles.at[b, i, :, pl.ds(0, 128)], tsem[b])

    def wait_stage(b):
        for i in range(4):
            pltpu.make_async_copy(
                tableT.at[pl.ds(0, 8), pl.ds(0, 128)],
                tiles.at[b, i, :, pl.ds(0, 128)], tsem[b]).wait()

    def transpose_block(b, nrows):
        def row8(r8, carry):
            vals = []
            for dr in range(8):
                rsp = lax.broadcast(r8 * 8 + dr, (16,))
                for h in range(2):
                    vals.append(
                        plsc.load_gather(tiles.at[b],
                                         [ivec[h], svec[h], rsp]))
            for dr in range(8):
                rr8 = r8 * 8 + dr
                for h in range(2):
                    blk[b, pl.ds(rr8 * D + h * 16, 16)] = vals[dr * 2 + h]
            return carry
        lax.fori_loop(0, nrows // 8, row8, 0)

    def start_write(jj, b):
        pltpu.async_copy(blk.at[b], aux.at[pl.ds(jj * 4096, 4096)], wsem[b])

    def wait_write(b):
        pltpu.make_async_copy(blk.at[b], aux.at[pl.ds(0, 4096)], wsem[b]).wait()

    stage(my_start, 0)
    stage(my_start + 1, 1)

    def pair(p, first):
        for b in range(2):
            k = p * 2 + b
            jj = my_start + k
            wait_stage(b)
            if not first:
                wait_write(b)
            transpose_block(b, 128)
            _maybe_when(k + 2 < ABLK, lambda b=b, jj=jj: stage(jj + 2, b))
            start_write(jj, b)

    pair(0, True)

    def loop_body(p, carry):
        pair(p, False)
        return carry

    lax.fori_loop(1, ABLK // 2, loop_body, 0)
    wait_write(0)
    wait_write(1)

    @pl.when(wid < 4)
    def _extra():
        jj = RT_FULL - 4 + wid
        stage(jj, 0)
        wait_stage(0)
        transpose_block(0, 128)
        start_write(jj, 0)
        wait_write(0)

    @pl.when(wid == NW - 1)
    def _tail():
        stage(wid - (NW - 1) + RT_FULL, 0)
        wait_stage(0)
        transpose_block(0, 64)
        pltpu.async_copy(blk.at[0, pl.ds(0, 2048)],
                         aux.at[pl.ds(RT_FULL * 4096, 2048)], ws0)
        pltpu.make_async_copy(blk.at[0, pl.ds(0, 2048)],
                              aux.at[pl.ds(RT_FULL * 4096, 2048)], ws0).wait()

    def group(g, carry):
        q = wid * 16 + g
        for j in range(4):
            pltpu.async_copy(
                idx_nat.at[pl.ds(q * 8, 8), pl.ds(j * 128, 128)],
                ibufa.at[j], qs)
        for j in range(4):
            pltpu.make_async_copy(
                idx_nat.at[pl.ds(0, 8), pl.ds(0, 128)],
                ibufa.at[j], qs).wait()
        for ap in range(8):
            for j in range(4):
                for seg in range(8):
                    ilin[pl.ds(ap * 512 + j * 128 + seg * 16, 16)] = (
                        ibufa[j, ap, pl.ds(seg * 16, 16)])
        pltpu.async_copy(ilin, idxlin.at[pl.ds(q * 4096, 4096)], qs)
        pltpu.make_async_copy(ilin, idxlin.at[pl.ds(q * 4096, 4096)], qs).wait()
        return carry

    lax.fori_loop(0, 16, group, 0)


def _gather_body(aux2d, idxlin, out, ibuf, g_buf, t_buf,
                 is0, is1, gs0, gs1, ws0, ws1):
    wid = lax.axis_index("s") * NC + lax.axis_index("c")
    isem = (is0, is1)
    gsem = (gs0, gs1)
    wsem = (ws0, ws1)

    base = wid * 128
    cc0 = jnp.arange(16, dtype=jnp.int32)

    def stage_idx(k, b):
        pltpu.async_copy(idxlin.at[pl.ds((base + k) * L_SEQ, L_SEQ)],
                         ibuf.at[b], isem[b])

    def wait_idx(b):
        pltpu.make_async_copy(idxlin.at[pl.ds(0, L_SEQ)],
                              ibuf.at[b], isem[b]).wait()

    def start_gather(b):
        pltpu.async_copy(aux2d.at[ibuf.at[b]], g_buf.at[b], gsem[b])

    def wait_gather(b):
        pltpu.make_async_copy(aux2d.at[ibuf.at[b]], g_buf.at[b],
                              gsem[b]).wait()

    def transpose_slab(b):
        def unit(v, carry):
            j = v // 8
            u0 = v % 8
            lv = cc0 + lax.broadcast(j * 128 + u0 * 16, (16,))
            for h in range(2):
                vals = [
                    plsc.load_gather(g_buf.at[b],
                                     [lv, lax.broadcast(c, (16,))])
                    for c in range(h * 16, h * 16 + 16)
                ]
                for i, c in enumerate(range(h * 16, h * 16 + 16)):
                    t_buf[b, (c // 8) * 4 + j, c % 8,
                          pl.ds(u0 * 16, 16)] = vals[i]
            return carry
        lax.fori_loop(0, 32, unit, 0)

    def start_write(k, b):
        pltpu.async_copy(t_buf.at[b], out.at[base + k], wsem[b])

    def wait_write(b):
        pltpu.make_async_copy(t_buf.at[b], out.at[base], wsem[b]).wait()

    stage_idx(0, 0)
    stage_idx(1, 1)
    wait_idx(0)
    start_gather(0)

    def pair(p, first):
        for b2 in range(2):
            k = p * 2 + b2
            wait_gather(b2)
            _maybe_when(k + 2 < 128, lambda b2=b2, k=k: stage_idx(k + 2, b2))

            def _next(b2=b2):
                wait_idx(1 - b2)
                start_gather(1 - b2)
            _maybe_when(k + 1 < 128, _next)
            if not first:
                wait_write(b2)
            transpose_slab(b2)
            start_write(k, b2)

    pair(0, True)

    def loop_body(p, carry):
        pair(p, False)
        return carry

    lax.fori_loop(1, 64, loop_body, 0)
    wait_write(0)
    wait_write(1)


def kernel(input_x, table):
    idx_nat = input_x.astype(jnp.int32)
    tableT = table.astype(jnp.float32).T

    mesh = plsc.VectorSubcoreMesh(core_axis_name="c", subcore_axis_name="s")

    fmt = functools.partial(
        pl.kernel,
        mesh=mesh,
        out_type=(
            jax.ShapeDtypeStruct((V * D,), jnp.float32),
            jax.ShapeDtypeStruct((B_BATCH * L_SEQ,), jnp.int32),
        ),
        scratch_types=[
            pltpu.VMEM((2, 4, 8, 131), jnp.float32),
            pltpu.VMEM((2, 4096), jnp.float32),
            pltpu.VMEM((4, 8, 128), jnp.int32),
            pltpu.VMEM((4096,), jnp.int32),
            pltpu.SemaphoreType.DMA,
            pltpu.SemaphoreType.DMA,
            pltpu.SemaphoreType.DMA,
            pltpu.SemaphoreType.DMA,
            pltpu.SemaphoreType.DMA,
        ],
        compiler_params=pltpu.CompilerParams(use_tc_tiling_on_sc=True, needs_layout_passes=False),
    )(_format_body)
    aux, idxlin = fmt(tableT, idx_nat)

    gat = functools.partial(
        pl.kernel,
        mesh=mesh,
        out_type=jax.ShapeDtypeStruct((B_BATCH, 16, 8, 128), jnp.float32),
        scratch_types=[
            pltpu.VMEM((2, L_SEQ), jnp.int32),
            pltpu.VMEM((2, L_SEQ, D), jnp.float32),
            pltpu.VMEM((2, 16, 8, 128), jnp.float32),
            pltpu.SemaphoreType.DMA,
            pltpu.SemaphoreType.DMA,
            pltpu.SemaphoreType.DMA,
            pltpu.SemaphoreType.DMA,
            pltpu.SemaphoreType.DMA,
            pltpu.SemaphoreType.DMA,
        ],
        compiler_params=pltpu.CompilerParams(use_tc_tiling_on_sc=False, needs_layout_passes=False),
    )(_gather_body)
    out_tiled = gat(aux.reshape(V, D), idxlin)

    out5 = out_tiled.reshape(B_BATCH, 4, 4, 8, 128)
    return out5.transpose(0, 2, 4, 1, 3).reshape(B_BATCH, L_SEQ, D)


# speedup vs baseline: 1.8360x; 1.0153x over previous
import functools

import jax
import jax.numpy as jnp
from jax import lax
from jax.experimental import pallas as pl
from jax.experimental.pallas import tpu as pltpu
from jax.experimental.pallas import tpu_sc as plsc

D = 32
NC = 2
NS = 16
NW = NC * NS
V = 1_000_000
RT = 7813
RT_FULL = 7812
ABLK = 244
B_BATCH = 4096
L_SEQ = 512


def _maybe_when(cond, fn):
    if isinstance(cond, bool):
        if cond:
            fn()
    else:
        pl.when(cond)(fn)


def _format_body(tableT, idx_nat, aux, idxlin,
                 tiles, blk, ibufa, ilin, ts0, ts1, ws0, ws1, qs):
    wid = lax.axis_index("s") * NC + lax.axis_index("c")
    tsem = (ts0, ts1)
    wsem = (ws0, ws1)

    cc0 = jnp.arange(16, dtype=jnp.int32)
    ivec = (cc0 // 8, (cc0 + 16) // 8)
    svec = (cc0 % 8, (cc0 + 16) % 8)

    my_start = wid * ABLK

    def stage(jj, b):
        for i in range(4):
            pltpu.async_copy(
                tableT.at[pl.ds(i * 8, 8), pl.ds(jj * 128, 128)],
                tiles.at[b, i, :, pl.ds(0, 128)], tsem[b])

    def wait_stage(b):
        for i in range(4):
            pltpu.make_async_copy(
                tableT.at[pl.ds(0, 8), pl.ds(0, 128)],
                tiles.at[b, i, :, pl.ds(0, 128)], tsem[b]).wait()

    def transpose_block(b, nrows):
        def row8(r8, carry):
            vals = []
            for dr in range(8):
                rsp = lax.broadcast(r8 * 8 + dr, (16,))
                for h in range(2):
                    vals.append(
                        plsc.load_gather(tiles.at[b],
                                         [ivec[h], svec[h], rsp]))
            for dr in range(8):
                rr8 = r8 * 8 + dr
                for h in range(2):
                    blk[b, pl.ds(rr8 * D + h * 16, 16)] = vals[dr * 2 + h]
            return carry
        lax.fori_loop(0, nrows // 8, row8, 0)

    def start_write(jj, b):
        pltpu.async_copy(blk.at[b], aux.at[pl.ds(jj * 4096, 4096)], wsem[b])

    def wait_write(b):
        pltpu.make_async_copy(blk.at[b], aux.at[pl.ds(0, 4096)], wsem[b]).wait()

    stage(my_start, 0)
    stage(my_start + 1, 1)

    def pair(p, first):
        for b in range(2):
            k = p * 2 + b
            jj = my_start + k
            wait_stage(b)
            if not first:
                wait_write(b)
            transpose_block(b, 128)
            _maybe_when(k + 2 < ABLK, lambda b=b, jj=jj: stage(jj + 2, b))
            start_write(jj, b)

    pair(0, True)

    def loop_body(p, carry):
        pair(p, False)
        return carry

    lax.fori_loop(1, ABLK // 2, loop_body, 0)
    wait_write(0)
    wait_write(1)

    @pl.when(wid < 4)
    def _extra():
        jj = RT_FULL - 4 + wid
        stage(jj, 0)
        wait_stage(0)
        transpose_block(0, 128)
        start_write(jj, 0)
        wait_write(0)

    @pl.when(wid == NW - 1)
    def _tail():
        stage(wid - (NW - 1) + RT_FULL, 0)
        wait_stage(0)
        transpose_block(0, 64)
        pltpu.async_copy(blk.at[0, pl.ds(0, 2048)],
                         aux.at[pl.ds(RT_FULL * 4096, 2048)], ws0)
        pltpu.make_async_copy(blk.at[0, pl.ds(0, 2048)],
                              aux.at[pl.ds(RT_FULL * 4096, 2048)], ws0).wait()

    def group(g, carry):
        q = wid * 16 + g
        for j in range(4):
            pltpu.async_copy(
                idx_nat.at[pl.ds(q * 8, 8), pl.ds(j * 128, 128)],
                ibufa.at[j], qs)
        for j in range(4):
            pltpu.make_async_copy(
                idx_nat.at[pl.ds(0, 8), pl.ds(0, 128)],
                ibufa.at[j], qs).wait()
        for ap in range(8):
            for j in range(4):
                for seg in range(8):
                    ilin[pl.ds(ap * 512 + j * 128 + seg * 16, 16)] = (
                        ibufa[j, ap, pl.ds(seg * 16, 16)])
        pltpu.async_copy(ilin, idxlin.at[pl.ds(q * 4096, 4096)], qs)
        pltpu.make_async_copy(ilin, idxlin.at[pl.ds(q * 4096, 4096)], qs).wait()
        return carry

    lax.fori_loop(0, 16, group, 0)


def _gather_body(aux2d, idxlin, out, ibuf, g_buf, t_buf,
                 is0, is1, gs0, gs1, ws0, ws1):
    wid = lax.axis_index("s") * NC + lax.axis_index("c")
    isem = (is0, is1)
    gsem = (gs0, gs1)
    wsem = (ws0, ws1)

    base = wid * 128
    cc0 = jnp.arange(16, dtype=jnp.int32)

    def stage_idx(k, b):
        pltpu.async_copy(idxlin.at[pl.ds((base + k) * L_SEQ, L_SEQ)],
                         ibuf.at[b], isem[b])

    def wait_idx(b):
        pltpu.make_async_copy(idxlin.at[pl.ds(0, L_SEQ)],
                              ibuf.at[b], isem[b]).wait()

    def start_gather(b):
        pltpu.async_copy(aux2d.at[ibuf.at[b]], g_buf.at[b], gsem[b])

    def wait_gather(b):
        pltpu.make_async_copy(aux2d.at[ibuf.at[b]], g_buf.at[b],
                              gsem[b]).wait()

    def transpose_slab(b):
        def unit(v, carry):
            j = v // 8
            u0 = v % 8
            lv = cc0 + lax.broadcast(j * 128 + u0 * 16, (16,))
            vals = [
                plsc.load_gather(g_buf.at[b],
                                 [lv, lax.broadcast(c, (16,))])
                for c in range(D)
            ]
            for c in range(D):
                t_buf[b, (c // 8) * 4 + j, c % 8,
                      pl.ds(u0 * 16, 16)] = vals[c]
            return carry
        lax.fori_loop(0, 32, unit, 0)

    def start_write(k, b):
        pltpu.async_copy(t_buf.at[b], out.at[base + k], wsem[b])

    def wait_write(b):
        pltpu.make_async_copy(t_buf.at[b], out.at[base], wsem[b]).wait()

    stage_idx(0, 0)
    stage_idx(1, 1)
    wait_idx(0)
    start_gather(0)

    def pair(p, first):
        for b2 in range(2):
            k = p * 2 + b2
            wait_gather(b2)
            _maybe_when(k + 2 < 128, lambda b2=b2, k=k: stage_idx(k + 2, b2))

            def _next(b2=b2):
                wait_idx(1 - b2)
                start_gather(1 - b2)
            _maybe_when(k + 1 < 128, _next)
            if not first:
                wait_write(b2)
            transpose_slab(b2)
            start_write(k, b2)

    pair(0, True)

    def loop_body(p, carry):
        pair(p, False)
        return carry

    lax.fori_loop(1, 64, loop_body, 0)
    wait_write(0)
    wait_write(1)


def kernel(input_x, table):
    idx_nat = input_x.astype(jnp.int32)
    tableT = table.astype(jnp.float32).T

    mesh = plsc.VectorSubcoreMesh(core_axis_name="c", subcore_axis_name="s")

    fmt = functools.partial(
        pl.kernel,
        mesh=mesh,
        out_type=(
            jax.ShapeDtypeStruct((V * D,), jnp.float32),
            jax.ShapeDtypeStruct((B_BATCH * L_SEQ,), jnp.int32),
        ),
        scratch_types=[
            pltpu.VMEM((2, 4, 8, 131), jnp.float32),
            pltpu.VMEM((2, 4096), jnp.float32),
            pltpu.VMEM((4, 8, 128), jnp.int32),
            pltpu.VMEM((4096,), jnp.int32),
            pltpu.SemaphoreType.DMA,
            pltpu.SemaphoreType.DMA,
            pltpu.SemaphoreType.DMA,
            pltpu.SemaphoreType.DMA,
            pltpu.SemaphoreType.DMA,
        ],
        compiler_params=pltpu.CompilerParams(use_tc_tiling_on_sc=True, needs_layout_passes=False),
    )(_format_body)
    aux, idxlin = fmt(tableT, idx_nat)

    gat = functools.partial(
        pl.kernel,
        mesh=mesh,
        out_type=jax.ShapeDtypeStruct((B_BATCH, 16, 8, 128), jnp.float32),
        scratch_types=[
            pltpu.VMEM((2, L_SEQ), jnp.int32),
            pltpu.VMEM((2, L_SEQ, D), jnp.float32),
            pltpu.VMEM((2, 16, 8, 128), jnp.float32),
            pltpu.SemaphoreType.DMA,
            pltpu.SemaphoreType.DMA,
            pltpu.SemaphoreType.DMA,
            pltpu.SemaphoreType.DMA,
            pltpu.SemaphoreType.DMA,
            pltpu.SemaphoreType.DMA,
        ],
        compiler_params=pltpu.CompilerParams(use_tc_tiling_on_sc=False, needs_layout_passes=False),
    )(_gather_body)
    out_tiled = gat(aux.reshape(V, D), idxlin)

    out5 = out_tiled.reshape(B_BATCH, 4, 4, 8, 128)
    return out5.transpose(0, 2, 4, 1, 3).reshape(B_BATCH, L_SEQ, D)
